# baseline (speedup 1.0000x reference)
import dataclasses
import functools

import jax
import jax.numpy as jnp
from jax import lax
from jax.experimental import pallas as pl
from jax.experimental.pallas import tpu as pltpu
from jax.experimental.pallas import tpu_sc as plsc

NC = 2
NS = 16
NW = NC * NS
LANES = 16
CHUNK = 128
RING = 8


def _compiler_params():
    cp = pltpu.CompilerParams()
    for field, val in (("needs_layout_passes", False),
                       ("use_tc_tiling_on_sc", True)):
        if field in pltpu.CompilerParams.__dataclass_fields__:
            cp = dataclasses.replace(cp, **{field: val})
    return cp


def kernel(user_ids, item_ids, user_factors, item_factors, user_biases,
           item_biases, global_bias):
    B = user_ids.shape[0]
    D = user_factors.shape[1]
    bpw = B // NW
    nchunks = bpw // CHUNK
    gpc = CHUNK // LANES

    uids = user_ids.astype(jnp.int32).reshape(B // CHUNK, CHUNK)
    iids = item_ids.astype(jnp.int32).reshape(B // CHUNK, CHUNK)
    uf_t = user_factors.T
    ub_t = user_biases.T
    ib_t = item_biases.T
    gb_vec = jnp.broadcast_to(global_bias.reshape(()), (LANES,))

    mesh = plsc.VectorSubcoreMesh(core_axis_name="c", subcore_axis_name="s")
    cp = _compiler_params()

    @functools.partial(
        pl.kernel,
        out_type=jax.ShapeDtypeStruct((B * D,), jnp.float32),
        mesh=mesh,
        compiler_params=cp,
        scratch_types=[
            pltpu.VMEM((nchunks, CHUNK), jnp.int32),
            pltpu.VMEM((RING, D, CHUNK), jnp.float32),
            pltpu.VMEM((CHUNK * D,), jnp.float32),
            pltpu.VMEM((CHUNK * D,), jnp.float32),
            pltpu.SemaphoreType.DMA,
        ] + [pltpu.SemaphoreType.DMA] * RING,
    )
    def gather_users(uid_hbm, uf_hbm, ug_hbm, uid_v, ublk, ustage0,
                     ustage1, sem_o, *usems):
        ustage = (ustage0, ustage1)
        wid = lax.axis_index("s") * NC + lax.axis_index("c")
        base = wid * bpw

        pltpu.sync_copy(uid_hbm.at[pl.ds(wid * nchunks, nchunks)], uid_v)
        iota16 = lax.iota(jnp.int32, LANES)

        def uid_at(r):
            grp = (r >> 4) << 4
            lane = r & 15
            vec = uid_v[r >> 7, pl.ds(grp & 127, LANES)]
            return jnp.sum(jnp.where(iota16 == lane, vec, 0))

        def fire_user(r, s):
            rid = uid_at(r)
            blk = pl.multiple_of((rid >> 7) << 7, CHUNK)
            pltpu.async_copy(uf_hbm.at[:, pl.ds(blk, CHUNK)],
                             ublk.at[s], usems[s])

        def drain_user(s):
            pltpu.make_async_copy(uf_hbm.at[:, pl.ds(0, CHUNK)],
                                  ublk.at[s], usems[s]).wait()

        for s in range(RING):
            fire_user(s, s)

        for j in range(nchunks):
            st = j % 2
            if j >= 2:
                pltpu.make_async_copy(
                    ug_hbm.at[pl.ds(0, CHUNK * D)],
                    ustage[st], sem_o).wait()

            @pl.loop(0, CHUNK, step=RING)
            def _(r):
                for s in range(RING):
                    rr = r + s
                    row = j * CHUNK + rr
                    drain_user(s)
                    rid = uid_at(row)
                    lane = jnp.full((LANES,), rid & 127, jnp.int32)
                    for t in range(D // LANES):
                        dsl = iota16 + t * LANES
                        pu = plsc.load_gather(ublk.at[s], [dsl, lane])
                        ustage[st][pl.ds(rr * D + t * LANES, LANES)] = pu

                    nxt = row + RING
                    @pl.when(nxt < bpw)
                    def _():
                        nid = uid_at(nxt)
                        blk = pl.multiple_of((nid >> 7) << 7, CHUNK)
                        pltpu.async_copy(uf_hbm.at[:, pl.ds(blk, CHUNK)],
                                         ublk.at[s], usems[s])

            pltpu.async_copy(
                ustage[st],
                ug_hbm.at[pl.ds((base + j * CHUNK) * D, CHUNK * D)], sem_o)

        for j in range(max(0, nchunks - 2), nchunks):
            pltpu.make_async_copy(
                ug_hbm.at[pl.ds(0, CHUNK * D)],
                ustage[j % 2], sem_o).wait()

    @functools.partial(
        pl.kernel,
        out_type=jax.ShapeDtypeStruct((B,), jnp.float32),
        mesh=mesh,
        compiler_params=cp,
        scratch_types=[
            pltpu.VMEM((nchunks, CHUNK), jnp.int32),
            pltpu.VMEM((nchunks, CHUNK), jnp.int32),
            pltpu.VMEM((CHUNK * D,), jnp.float32),
            pltpu.VMEM((CHUNK * D,), jnp.float32),
            pltpu.VMEM((2, CHUNK, D), jnp.float32),
            pltpu.VMEM((bpw,), jnp.float32),
            pltpu.VMEM((bpw,), jnp.float32),
            pltpu.VMEM((bpw,), jnp.float32),
            pltpu.VMEM((LANES,), jnp.float32),
            pltpu.SemaphoreType.DMA,
            pltpu.SemaphoreType.DMA,
            pltpu.SemaphoreType.DMA,
            pltpu.SemaphoreType.DMA,
            pltpu.SemaphoreType.DMA,
        ],
    )
    def combine(uid_hbm, iid_hbm, ug_hbm, if_hbm, ub_hbm, ib_hbm, gb_hbm,
                out_hbm, uid_v, iid_v, us0, us1, irows, ubv, ibv, outv,
                gbv, sem_b, su0, su1, si0, si1):
        us = (us0, us1)
        wid = lax.axis_index("s") * NC + lax.axis_index("c")
        base = wid * bpw
        usems = (su0, su1)
        isems = (si0, si1)

        pltpu.sync_copy(uid_hbm.at[pl.ds(wid * nchunks, nchunks)], uid_v)
        pltpu.sync_copy(iid_hbm.at[pl.ds(wid * nchunks, nchunks)], iid_v)
        pltpu.sync_copy(gb_hbm, gbv)
        iota16 = lax.iota(jnp.int32, LANES)

        def fire_users(j, slot):
            pltpu.async_copy(
                ug_hbm.at[pl.ds((base + j * CHUNK) * D, CHUNK * D)],
                us[slot], usems[slot])

        def drain_users(slot):
            pltpu.make_async_copy(ug_hbm.at[pl.ds(0, CHUNK * D)],
                                  us[slot], usems[slot]).wait()

        def fire_items(j, slot):
            @pl.loop(0, gpc)
            def _(g):
                ivec = iid_v[j, pl.ds(g * LANES, LANES)]
                for lane in range(LANES):
                    rid = jnp.sum(jnp.where(iota16 == lane, ivec, 0))
                    dst = pl.ds(g * LANES + lane, 1)
                    pltpu.async_copy(if_hbm.at[pl.ds(rid, 1)],
                                     irows.at[slot].at[dst], isems[slot])

        def drain_items(slot):
            pltpu.make_async_copy(if_hbm.at[pl.ds(0, CHUNK)],
                                  irows.at[slot], isems[slot]).wait()

        bias_copies = []
        for j in range(nchunks):
            sl = pl.ds(j * CHUNK, CHUNK)
            bias_copies.append(
                pltpu.async_copy(ub_hbm.at[0].at[uid_v.at[j]],
                                 ubv.at[sl], sem_b))
            bias_copies.append(
                pltpu.async_copy(ib_hbm.at[0].at[iid_v.at[j]],
                                 ibv.at[sl], sem_b))

        fire_items(0, 0)
        fire_users(0, 0)
        for cp_ in bias_copies:
            cp_.wait()

        gb = gbv[...]

        for j in range(nchunks):
            slot = j % 2
            drain_items(slot)
            drain_users(slot)
            if j + 1 < nchunks:
                fire_items(j + 1, 1 - slot)
                fire_users(j + 1, 1 - slot)
            u_slot = us[slot]
            i_slot = irows.at[slot]

            @pl.loop(0, gpc)
            def _(g):
                osl = pl.ds(j * CHUNK + g * LANES, LANES)
                row16 = iota16 + g * LANES
                ubase = row16 * D
                acc = ubv[osl] + ibv[osl] + gb
                for d in range(D):
                    cd = jnp.full((LANES,), d, jnp.int32)
                    pu = plsc.load_gather(u_slot, [ubase + d])
                    pv = plsc.load_gather(i_slot, [row16, cd])
                    acc = acc + pu * pv
                outv[osl] = acc

        pltpu.sync_copy(outv, out_hbm.at[pl.ds(base, bpw)])

    ugath = gather_users(uids, uf_t)
    return combine(uids, iids, ugath, item_factors, ub_t, ib_t, gb_vec)


# speedup vs baseline: 1.1271x; 1.1271x over previous
import dataclasses
import functools

import jax
import jax.numpy as jnp
from jax import lax
from jax.experimental import pallas as pl
from jax.experimental.pallas import tpu as pltpu
from jax.experimental.pallas import tpu_sc as plsc

NC = 2
NS = 16
NW = NC * NS
LANES = 16
CHUNK = 128
RING = 8


def kernel(user_ids, item_ids, user_factors, item_factors, user_biases,
           item_biases, global_bias):
    B = user_ids.shape[0]
    D = user_factors.shape[1]
    bpw = B // NW
    nchunks = bpw // CHUNK
    gpc = CHUNK // LANES

    uids = user_ids.astype(jnp.int32).reshape(B // CHUNK, CHUNK)
    iids = item_ids.astype(jnp.int32).reshape(B // CHUNK, CHUNK)
    uf_t = user_factors.T
    ub_t = user_biases.T
    ib_t = item_biases.T
    gb_vec = jnp.broadcast_to(global_bias.reshape(()), (LANES,))

    mesh = plsc.VectorSubcoreMesh(core_axis_name="c", subcore_axis_name="s")

    cp = pltpu.CompilerParams()
    for field, val in (("needs_layout_passes", False),
                       ("use_tc_tiling_on_sc", True)):
        if field in pltpu.CompilerParams.__dataclass_fields__:
            cp = dataclasses.replace(cp, **{field: val})

    @functools.partial(
        pl.kernel,
        out_type=jax.ShapeDtypeStruct((B,), jnp.float32),
        mesh=mesh,
        compiler_params=cp,
        scratch_types=[
            pltpu.VMEM((nchunks, CHUNK), jnp.int32),
            pltpu.VMEM((nchunks, CHUNK), jnp.int32),
            pltpu.VMEM((RING, D, CHUNK), jnp.float32),
            pltpu.VMEM((2, CHUNK, D), jnp.float32),
            pltpu.VMEM((bpw,), jnp.float32),
            pltpu.VMEM((bpw,), jnp.float32),
            pltpu.VMEM((bpw,), jnp.float32),
            pltpu.VMEM((LANES,), jnp.float32),
            pltpu.SemaphoreType.DMA,
            pltpu.SemaphoreType.DMA,
            pltpu.SemaphoreType.DMA,
        ] + [pltpu.SemaphoreType.DMA] * RING,
    )
    def mf_kernel(uid_hbm, iid_hbm, uf_hbm, if_hbm, ub_hbm, ib_hbm, gb_hbm,
                  out_hbm, uid_v, iid_v, ublk, irows, ubv, ibv, outv, gbv,
                  sem_b, sem_i0, sem_i1, *usems):
        wid = lax.axis_index("s") * NC + lax.axis_index("c")
        base = wid * bpw
        isems = (sem_i0, sem_i1)

        idx_rows = pl.ds(wid * nchunks, nchunks)
        pltpu.sync_copy(uid_hbm.at[idx_rows], uid_v)
        pltpu.sync_copy(iid_hbm.at[idx_rows], iid_v)
        pltpu.sync_copy(gb_hbm, gbv)

        bias_copies = []
        for j in range(nchunks):
            sl = pl.ds(j * CHUNK, CHUNK)
            bias_copies.append(
                pltpu.async_copy(ub_hbm.at[0].at[uid_v.at[j]],
                                 ubv.at[sl], sem_b))
            bias_copies.append(
                pltpu.async_copy(ib_hbm.at[0].at[iid_v.at[j]],
                                 ibv.at[sl], sem_b))

        iota16 = lax.iota(jnp.int32, LANES)
        zeros16 = jnp.zeros((LANES,), jnp.int32)

        def uid_at(r):
            grp = (r >> 4) << 4
            lane = r & 15
            vec = uid_v[r >> 7, pl.ds(grp & 127, LANES)]
            return jnp.sum(jnp.where(iota16 == lane, vec, 0))

        def fire_user(r, s):
            rid = uid_at(r)
            blk = pl.multiple_of((rid >> 7) << 7, CHUNK)
            pltpu.async_copy(uf_hbm.at[:, pl.ds(blk, CHUNK)],
                             ublk.at[s], usems[s])

        def drain_user(s):
            pltpu.make_async_copy(uf_hbm.at[:, pl.ds(0, CHUNK)],
                                  ublk.at[s], usems[s]).wait()

        def fire_items(j, slot):
            @pl.loop(0, gpc)
            def _(g):
                ivec = iid_v[j, pl.ds(g * LANES, LANES)]
                for lane in range(LANES):
                    rid = jnp.sum(jnp.where(iota16 == lane, ivec, 0))
                    dst = pl.ds(g * LANES + lane, 1)
                    pltpu.async_copy(if_hbm.at[pl.ds(rid, 1)],
                                     irows.at[slot].at[dst], isems[slot])

        def drain_items(slot):
            pltpu.make_async_copy(if_hbm.at[pl.ds(0, CHUNK)],
                                  irows.at[slot], isems[slot]).wait()

        fire_items(0, 0)
        for cp_ in bias_copies:
            cp_.wait()
        gb = gbv[...]

        for s in range(RING):
            fire_user(s, s)

        for j in range(nchunks):
            slot = j % 2
            drain_items(slot)
            if j + 1 < nchunks:
                fire_items(j + 1, 1 - slot)
            i_slot = irows.at[slot]

            @pl.loop(0, CHUNK, step=RING)
            def _(r):
                for s in range(RING):
                    rr = r + s
                    row = j * CHUNK + rr
                    drain_user(s)
                    rid = uid_at(row)
                    lane = jnp.full((LANES,), rid & 127, jnp.int32)
                    acc = jnp.zeros((LANES,), jnp.float32)
                    for t in range(D // LANES):
                        dsl = iota16 + t * LANES
                        pu = plsc.load_gather(ublk.at[s], [dsl, lane])
                        pv = i_slot[rr, pl.ds(t * LANES, LANES)]
                        acc = acc + pu * pv
                    dot = jnp.sum(acc)
                    row16 = jnp.full((LANES,), row, jnp.int32)
                    ub16 = plsc.load_gather(ubv, [row16])
                    ib16 = plsc.load_gather(ibv, [row16])
                    val = dot + ub16 + ib16 + gb
                    plsc.store_scatter(outv, [row16], val,
                                       mask=iota16 == 0)

                    nxt = row + RING
                    @pl.when(nxt < bpw)
                    def _():
                        nid = uid_at(nxt)
                        blk = pl.multiple_of((nid >> 7) << 7, CHUNK)
                        pltpu.async_copy(uf_hbm.at[:, pl.ds(blk, CHUNK)],
                                         ublk.at[s], usems[s])

        pltpu.sync_copy(outv, out_hbm.at[pl.ds(base, bpw)])

    return mf_kernel(uids, iids, uf_t, item_factors, ub_t,
                     ib_t, gb_vec)
